# baseline (speedup 1.0000x reference)
import functools

import jax
import jax.numpy as jnp
from jax import lax
from jax.experimental import pallas as pl
from jax.experimental.pallas import tpu as pltpu
from jax.experimental.pallas import tpu_sc as plsc

D = 32
B = 16384
NC = 2
NS = 16
L = 16
NW = NC * NS
BPW = B // NW
CHUNK = 128
NCHUNK = BPW // CHUNK


def _sc_body(w_hbm, idx0_hbm, idx1_hbm, out_hbm,
             idx0_v, idx1_v, rows0_v, rows1_v, out_v, sem):
    wid = lax.axis_index("s") * NC + lax.axis_index("c")
    crow = wid * NCHUNK
    pltpu.sync_copy(idx0_hbm.at[pl.ds(crow, NCHUNK)], idx0_v)
    pltpu.sync_copy(idx1_hbm.at[pl.ds(crow, NCHUNK)], idx1_v)

    copies = []
    for j in range(NCHUNK):
        copies.append(pltpu.async_copy(
            w_hbm.at[idx0_v.at[j]], rows0_v.at[pl.ds(j * CHUNK, CHUNK)], sem))
        copies.append(pltpu.async_copy(
            w_hbm.at[idx1_v.at[j]], rows1_v.at[pl.ds(j * CHUNK, CHUNK)], sem))
    for c in copies:
        c.wait()

    lanes = lax.iota(jnp.int32, L)

    def block_body(b, carry):
        rows = b * L + lanes
        acc = jnp.zeros((L,), jnp.float32)
        for d in range(D):
            col = jnp.full((L,), d, jnp.int32)
            a0 = plsc.load_gather(rows0_v, [rows, col])
            a1 = plsc.load_gather(rows1_v, [rows, col])
            acc = acc + a0 * a1
        out_v[pl.ds(b * L, L)] = acc
        return carry

    lax.fori_loop(0, BPW // L, block_body, 0)
    pltpu.sync_copy(out_v, out_hbm.at[pl.ds(wid * BPW, BPW)])


@jax.jit
def kernel(sparse_features, W):
    idx = sparse_features.astype(jnp.int32)
    idx0 = idx[:, 0].reshape(B // CHUNK, CHUNK)
    idx1 = idx[:, 1].reshape(B // CHUNK, CHUNK)
    mesh = plsc.VectorSubcoreMesh(core_axis_name="c", subcore_axis_name="s")
    out = pl.kernel(
        _sc_body,
        out_type=jax.ShapeDtypeStruct((B,), jnp.float32),
        mesh=mesh,
        compiler_params=pltpu.CompilerParams(
            needs_layout_passes=False, use_tc_tiling_on_sc=False),
        scratch_types=[
            pltpu.VMEM((NCHUNK, CHUNK), jnp.int32),
            pltpu.VMEM((NCHUNK, CHUNK), jnp.int32),
            pltpu.VMEM((BPW, D), jnp.float32),
            pltpu.VMEM((BPW, D), jnp.float32),
            pltpu.VMEM((BPW,), jnp.float32),
            pltpu.SemaphoreType.DMA,
        ],
    )(W, idx0, idx1)
    return out.reshape(B, 1)
